# baseline (speedup 1.0000x reference)
import jax
import jax.numpy as jnp
from jax.experimental import pallas as pl
from jax.experimental.pallas import tpu as pltpu

_RADIUS = 2
_WIN = 2 * _RADIUS + 1
_CHUNK_H = 28
_NCHUNK = 8


def _make_body(B, H, W, C):
    Cout = C - 2

    def _body(idx_ref, fields_ref, out_ref, winbuf, buf, wsem, osem):
        b = pl.program_id(0)
        nb = pl.num_programs(0)

        def window_copy(bb):
            py = idx_ref[bb, 0]
            px = idx_ref[bb, 1]
            return pltpu.make_async_copy(
                fields_ref.at[
                    bb, pl.ds(py - _RADIUS, _WIN), pl.ds(px - _RADIUS, _WIN), :
                ],
                winbuf.at[bb],
                wsem,
            )

        def chunk_copy(bb, c):
            return pltpu.make_async_copy(
                buf.at[c],
                out_ref.at[bb, pl.ds(c * _CHUNK_H, _CHUNK_H), :, :],
                osem.at[c],
            )

        @pl.when(b == 0)
        def _():
            for bb in range(B):
                window_copy(bb).start()
            for bb in range(B):
                window_copy(bb).wait()

        @pl.when(b >= 1)
        def _():
            for c in range(_NCHUNK):
                chunk_copy(b - 1, c).wait()

        w = winbuf[b, :, :, 2:]
        mean = jnp.sum(w, axis=(0, 1)) * (1.0 / (_WIN * _WIN))
        bc = jnp.broadcast_to(mean[None, None, :], (_CHUNK_H, W, Cout))
        for c in range(_NCHUNK):
            buf[c] = bc

        for c in range(_NCHUNK):
            chunk_copy(b, c).start()

        @pl.when(b == nb - 1)
        def _():
            for c in range(_NCHUNK):
                chunk_copy(b, c).wait()

    return _body


def kernel(fields, pump_indices):
    B, H, W, C = fields.shape
    Cout = C - 2
    idx = pump_indices.astype(jnp.int32)

    grid_spec = pltpu.PrefetchScalarGridSpec(
        num_scalar_prefetch=1,
        grid=(B,),
        in_specs=[
            pl.BlockSpec(memory_space=pl.ANY),
        ],
        out_specs=pl.BlockSpec(memory_space=pl.ANY),
        scratch_shapes=[
            pltpu.VMEM((B, _WIN, _WIN, C), jnp.float32),
            pltpu.VMEM((_NCHUNK, _CHUNK_H, W, Cout), jnp.float32),
            pltpu.SemaphoreType.DMA,
            pltpu.SemaphoreType.DMA((_NCHUNK,)),
        ],
    )

    return pl.pallas_call(
        _make_body(B, H, W, C),
        grid_spec=grid_spec,
        out_shape=jax.ShapeDtypeStruct((B, H, W, Cout), jnp.float32),
    )(idx, fields)


# speedup vs baseline: 1.0075x; 1.0075x over previous
import functools

import jax
import jax.numpy as jnp
from jax import lax
from jax.experimental import pallas as pl
from jax.experimental.pallas import tpu as pltpu
from jax.experimental.pallas import tpu_sc as plsc

_RADIUS = 2
_WIN = 2 * _RADIUS + 1
_ROWS = 4


def _mean_body(B, C, idx_ref, fields_ref, out_ref, win_ref, sem):
    def window_copy(bb):
        py = idx_ref[bb, 0]
        px = idx_ref[bb, 1]
        return pltpu.make_async_copy(
            fields_ref.at[
                bb, pl.ds(py - _RADIUS, _WIN), pl.ds(px - _RADIUS, _WIN), :
            ],
            win_ref.at[bb],
            sem,
        )

    for bb in range(B):
        window_copy(bb).start()
    for bb in range(B):
        window_copy(bb).wait()
        m = jnp.sum(win_ref[bb, :, :, 2:], axis=(0, 1)) * (1.0 / (_WIN * _WIN))
        out_ref[bb] = jnp.broadcast_to(
            m[None, None, :], out_ref.shape[1:]
        )


def _sc_broadcast_body(H, tmpl_ref, out_ref, rep, sem):
    b = lax.axis_index("s")
    half = lax.axis_index("c")

    pltpu.sync_copy(tmpl_ref.at[b], rep)

    rows_half = H // 2
    nchunk = rows_half // _ROWS
    base = half * rows_half
    copies = [
        pltpu.async_copy(
            rep, out_ref.at[b, pl.ds(base + _ROWS * k, _ROWS), :, :], sem
        )
        for k in range(nchunk)
    ]
    for cp in copies:
        cp.wait()


def kernel(fields, pump_indices):
    B, H, W, C = fields.shape
    Cout = C - 2
    idx = pump_indices.astype(jnp.int32)

    mean_grid = pltpu.PrefetchScalarGridSpec(
        num_scalar_prefetch=1,
        grid=(1,),
        in_specs=[pl.BlockSpec(memory_space=pl.ANY)],
        out_specs=pl.BlockSpec((B, _ROWS, W, Cout), lambda i, idx_ref: (0, 0, 0, 0)),
        scratch_shapes=[
            pltpu.VMEM((B, _WIN, _WIN, C), jnp.float32),
            pltpu.SemaphoreType.DMA,
        ],
    )
    tmpl = pl.pallas_call(
        functools.partial(_mean_body, B, C),
        grid_spec=mean_grid,
        out_shape=jax.ShapeDtypeStruct((B, _ROWS, W, Cout), jnp.float32),
    )(idx, fields)

    mesh = plsc.VectorSubcoreMesh(core_axis_name="c", subcore_axis_name="s")
    sc_fn = pl.kernel(
        functools.partial(_sc_broadcast_body, H),
        out_type=jax.ShapeDtypeStruct((B, H, W, Cout), jnp.float32),
        mesh=mesh,
        scratch_types=[
            pltpu.VMEM((_ROWS, W, Cout), jnp.float32),
            pltpu.SemaphoreType.DMA,
        ],
    )
    return sc_fn(tmpl)
